# Initial kernel scaffold; baseline (speedup 1.0000x reference)
import jax
import jax.numpy as jnp
from jax.experimental import pallas as pl


def kernel(x, edge_index, x_initial, x_lead, W1n, b1n, W1g, W1i, W1l, W2n, b2n, W2g, W2i, W2l, W3n, b3n, W3g, W3i, W3l, Wfc1, bfc1, Wfc2, bfc2):
    raise NotImplementedError("write your pallas kernel here")


# speedup vs baseline: 2.8641x; 2.8641x over previous
import functools

import jax
import jax.numpy as jnp
from jax import lax
from jax.experimental import pallas as pl
from jax.experimental.pallas import tpu as pltpu
from jax.experimental.pallas import tpu_sc as plsc

N = 10000
E = 320000
D = 128
A = 64

NC = 2
NS = 16
NW = NC * NS
CH = 128
K = 80
KB = 16
NKB = K // KB
E_PAD = NW * K * CH
N_ACC = 10240

ROWS_BLK = 2000
N_BLKS = N // ROWS_BLK


_sc_mesh = plsc.VectorSubcoreMesh(
    core_axis_name="c", subcore_axis_name="s", num_cores=NC, num_subcores=NS)


@functools.partial(
    pl.kernel,
    out_type=jax.ShapeDtypeStruct((NC, N_ACC, D), jnp.float32),
    mesh=_sc_mesh,
    scratch_types=[
        pltpu.MemorySpace.VMEM_SHARED((N_ACC, D), jnp.float32),
        pltpu.VMEM((KB, CH), jnp.int32),
        pltpu.VMEM((KB, CH), jnp.int32),
        pltpu.VMEM((CH, D), jnp.float32),
        pltpu.SemaphoreType.DMA,
    ],
)
def _sc_segsum(table, src_idx, dst_idx, out, acc, src_v, dst_v, rows, sem):
    c = lax.axis_index("c")
    s = lax.axis_index("s")
    w = c * NS + s

    zero = jnp.zeros((16,), jnp.float32)

    def _zrow(i, carry):
        for jj in range(D // 16):
            rows[i, pl.ds(jj * 16, 16)] = zero
        return carry

    lax.fori_loop(0, CH, _zrow, 0)
    for b in range(5):
        pltpu.sync_copy(rows, acc.at[pl.ds(s * 640 + b * CH, CH)])
    plsc.subcore_barrier()

    def _idx_block(kb, carry):
        pltpu.sync_copy(src_idx.at[w].at[pl.ds(kb * KB, KB)], src_v)
        pltpu.sync_copy(dst_idx.at[w].at[pl.ds(kb * KB, KB)], dst_v)

        def _edge_chunk(j, carry2):
            pltpu.async_copy(table.at[src_v.at[j]], rows, sem).wait()
            pltpu.sync_copy(rows, acc.at[dst_v.at[j]], add=True)
            return carry2

        lax.fori_loop(0, KB, _edge_chunk, 0)
        return carry

    lax.fori_loop(0, NKB, _idx_block, 0)
    plsc.subcore_barrier()

    pltpu.sync_copy(acc.at[pl.ds(s * 640, 640)],
                    out.at[c].at[pl.ds(s * 640, 640)])


def _colsum3_body(x_ref, xi_ref, xl_ref, sx_ref, si_ref, sl_ref):
    @pl.when(pl.program_id(0) == 0)
    def _init():
        sx_ref[...] = jnp.zeros_like(sx_ref)
        si_ref[...] = jnp.zeros_like(si_ref)
        sl_ref[...] = jnp.zeros_like(sl_ref)

    sx_ref[...] += jnp.sum(x_ref[...], axis=0, keepdims=True)
    si_ref[...] += jnp.sum(xi_ref[...], axis=0, keepdims=True)
    sl_ref[...] += jnp.sum(xl_ref[...], axis=0, keepdims=True)


_row_spec = pl.BlockSpec((ROWS_BLK, D), lambda i: (i, 0))
_vec_spec = pl.BlockSpec((1, D), lambda i: (0, 0))
_mat_spec = pl.BlockSpec((D, D), lambda i: (0, 0))
_agg_spec = pl.BlockSpec((NC, ROWS_BLK, D), lambda i: (0, i, 0))

_colsum3 = pl.pallas_call(
    _colsum3_body,
    grid=(N_BLKS,),
    in_specs=[_row_spec, _row_spec, _row_spec],
    out_specs=[_vec_spec, _vec_spec, _vec_spec],
    out_shape=[jax.ShapeDtypeStruct((1, D), jnp.float32)] * 3,
)


def _dense_body(x_ref, agg_ref, hg_ref, hi_ref, hl_ref, Wn_ref, bn_ref,
                Wg_ref, Wi_ref, Wl_ref, h_ref, hsum_ref):
    rv = (bn_ref[...] + hg_ref[...] @ Wg_ref[...] + hi_ref[...] @ Wi_ref[...]
          + hl_ref[...] @ Wl_ref[...])
    t = x_ref[...] + agg_ref[0] + agg_ref[1]
    h = jnp.maximum(t @ Wn_ref[...] + rv, 0.0)
    h_ref[...] = h

    @pl.when(pl.program_id(0) == 0)
    def _init():
        hsum_ref[...] = jnp.zeros_like(hsum_ref)

    hsum_ref[...] += jnp.sum(h, axis=0, keepdims=True)


_dense = pl.pallas_call(
    _dense_body,
    grid=(N_BLKS,),
    in_specs=[_row_spec, _agg_spec, _vec_spec, _vec_spec, _vec_spec,
              _mat_spec, _vec_spec, _mat_spec, _mat_spec, _mat_spec],
    out_specs=[_row_spec, _vec_spec],
    out_shape=[jax.ShapeDtypeStruct((N, D), jnp.float32),
               jax.ShapeDtypeStruct((1, D), jnp.float32)],
)


def _dense3_body(x_ref, agg_ref, hg_ref, hi_ref, hl_ref, Wn_ref, bn_ref,
                 Wg_ref, Wi_ref, Wl_ref, Wfc1_ref, bfc1_ref, Wfc2_ref,
                 bfc2_ref, out_ref, hsum_ref):
    rv = (bn_ref[...] + hg_ref[...] @ Wg_ref[...] + hi_ref[...] @ Wi_ref[...]
          + hl_ref[...] @ Wl_ref[...])
    t = x_ref[...] + agg_ref[0] + agg_ref[1]
    h = jnp.maximum(t @ Wn_ref[...] + rv, 0.0)

    @pl.when(pl.program_id(0) == 0)
    def _init():
        hsum_ref[...] = jnp.zeros_like(hsum_ref)

    hsum_ref[...] += jnp.sum(h, axis=0, keepdims=True)

    @pl.when(pl.program_id(0) == N_BLKS - 1)
    def _head():
        z = jnp.maximum(hsum_ref[...] @ Wfc1_ref[...] + bfc1_ref[...], 0.0)
        out_ref[...] = z @ Wfc2_ref[...] + bfc2_ref[...]


_dense3 = pl.pallas_call(
    _dense3_body,
    grid=(N_BLKS,),
    in_specs=[_row_spec, _agg_spec, _vec_spec, _vec_spec, _vec_spec,
              _mat_spec, _vec_spec, _mat_spec, _mat_spec, _mat_spec,
              _mat_spec, _vec_spec, pl.BlockSpec((D, A), lambda i: (0, 0)),
              pl.BlockSpec((1, A), lambda i: (0, 0))],
    out_specs=pl.BlockSpec((1, A), lambda i: (0, 0)),
    out_shape=jax.ShapeDtypeStruct((1, A), jnp.float32),
    scratch_shapes=[pltpu.VMEM((1, D), jnp.float32)],
)


def kernel(x, edge_index, x_initial, x_lead, W1n, b1n, W1g, W1i, W1l, W2n,
           b2n, W2g, W2i, W2l, W3n, b3n, W3g, W3i, W3l, Wfc1, bfc1, Wfc2,
           bfc2):
    src = edge_index[0]
    dst = edge_index[1]
    pad = E_PAD - E
    src_p = jnp.concatenate([src, jnp.zeros((pad,), jnp.int32)])
    dst_p = jnp.concatenate([dst, jnp.full((pad,), N, jnp.int32)])
    src_p = src_p.reshape(NW, K, CH)
    dst_p = dst_p.reshape(NW, K, CH)

    b1n_ = b1n.reshape(1, D)
    b2n_ = b2n.reshape(1, D)
    b3n_ = b3n.reshape(1, D)
    bfc1_ = bfc1.reshape(1, D)
    bfc2_ = bfc2.reshape(1, A)

    sx, si, sl = _colsum3(x, x_initial, x_lead)

    agg1 = _sc_segsum(x, src_p, dst_p)
    h1, s1 = _dense(x, agg1, sx, si, sl, W1n, b1n_, W1g, W1i, W1l)
    agg2 = _sc_segsum(h1, src_p, dst_p)
    h2, s2 = _dense(h1, agg2, s1, si, sl, W2n, b2n_, W2g, W2i, W2l)
    agg3 = _sc_segsum(h2, src_p, dst_p)
    return _dense3(h2, agg3, s2, si, sl, W3n, b3n_, W3g, W3i, W3l, Wfc1,
                   bfc1_, Wfc2, bfc2_)


# speedup vs baseline: 3.2371x; 1.1302x over previous
import functools

import jax
import jax.numpy as jnp
from jax import lax
from jax.experimental import pallas as pl
from jax.experimental.pallas import tpu as pltpu
from jax.experimental.pallas import tpu_sc as plsc

N = 10000
E = 320000
D = 128
A = 64

NC = 2
NS = 16
NW = NC * NS
CH = 128
K = 80
KB = 40
NKB = K // KB
E_PAD = NW * K * CH
N_ACC = 10240

ROWS_BLK = 2000
N_BLKS = N // ROWS_BLK


_sc_mesh = plsc.VectorSubcoreMesh(
    core_axis_name="c", subcore_axis_name="s", num_cores=NC, num_subcores=NS)


@functools.partial(
    pl.kernel,
    out_type=jax.ShapeDtypeStruct((NC, N_ACC, D), jnp.float32),
    mesh=_sc_mesh,
    scratch_types=[
        pltpu.MemorySpace.VMEM_SHARED((N_ACC, D), jnp.float32),
        pltpu.VMEM((KB, CH), jnp.int32),
        pltpu.VMEM((KB, CH), jnp.int32),
        pltpu.VMEM((CH, D), jnp.float32),
        pltpu.VMEM((CH, D), jnp.float32),
        pltpu.SemaphoreType.DMA,
        pltpu.SemaphoreType.DMA,
    ],
)
def _sc_segsum(table, src_idx, dst_idx, out, acc, src_v, dst_v, rows0, rows1,
               sem0, sem1):
    c = lax.axis_index("c")
    s = lax.axis_index("s")
    w = c * NS + s

    zero = jnp.zeros((16,), jnp.float32)

    def _zrow(i, carry):
        for jj in range(D // 16):
            rows0[i, pl.ds(jj * 16, 16)] = zero
        return carry

    lax.fori_loop(0, CH, _zrow, 0)
    for b in range(5):
        pltpu.sync_copy(rows0, acc.at[pl.ds(s * 640 + b * CH, CH)])
    plsc.subcore_barrier()

    def _wait(sem, buf):
        pltpu.make_async_copy(table.at[pl.ds(0, CH)], buf, sem).wait()

    for kb in range(NKB):
        pltpu.sync_copy(src_idx.at[w].at[pl.ds(kb * KB, KB)], src_v)
        pltpu.sync_copy(dst_idx.at[w].at[pl.ds(kb * KB, KB)], dst_v)

        pltpu.async_copy(table.at[src_v.at[0]], rows0, sem0)

        def _pair(t, carry2):
            j = 2 * t
            pltpu.async_copy(table.at[src_v.at[j + 1]], rows1, sem1)
            _wait(sem0, rows0)
            pltpu.sync_copy(rows0, acc.at[dst_v.at[j]], add=True)

            @pl.when(t < KB // 2 - 1)
            def _arm():
                pltpu.async_copy(table.at[src_v.at[j + 2]], rows0, sem0)

            _wait(sem1, rows1)
            pltpu.sync_copy(rows1, acc.at[dst_v.at[j + 1]], add=True)
            return carry2

        lax.fori_loop(0, KB // 2, _pair, 0)
    plsc.subcore_barrier()

    pltpu.sync_copy(acc.at[pl.ds(s * 640, 640)],
                    out.at[c].at[pl.ds(s * 640, 640)])


def _colsum3_body(x_ref, xi_ref, xl_ref, sx_ref, si_ref, sl_ref):
    @pl.when(pl.program_id(0) == 0)
    def _init():
        sx_ref[...] = jnp.zeros_like(sx_ref)
        si_ref[...] = jnp.zeros_like(si_ref)
        sl_ref[...] = jnp.zeros_like(sl_ref)

    sx_ref[...] += jnp.sum(x_ref[...], axis=0, keepdims=True)
    si_ref[...] += jnp.sum(xi_ref[...], axis=0, keepdims=True)
    sl_ref[...] += jnp.sum(xl_ref[...], axis=0, keepdims=True)


_row_spec = pl.BlockSpec((ROWS_BLK, D), lambda i: (i, 0))
_vec_spec = pl.BlockSpec((1, D), lambda i: (0, 0))
_mat_spec = pl.BlockSpec((D, D), lambda i: (0, 0))
_agg_spec = pl.BlockSpec((NC, ROWS_BLK, D), lambda i: (0, i, 0))

_colsum3 = pl.pallas_call(
    _colsum3_body,
    grid=(N_BLKS,),
    in_specs=[_row_spec, _row_spec, _row_spec],
    out_specs=[_vec_spec, _vec_spec, _vec_spec],
    out_shape=[jax.ShapeDtypeStruct((1, D), jnp.float32)] * 3,
)


def _dense_body(x_ref, agg_ref, hg_ref, hi_ref, hl_ref, Wn_ref, bn_ref,
                Wg_ref, Wi_ref, Wl_ref, h_ref, hsum_ref):
    rv = (bn_ref[...] + hg_ref[...] @ Wg_ref[...] + hi_ref[...] @ Wi_ref[...]
          + hl_ref[...] @ Wl_ref[...])
    t = x_ref[...] + agg_ref[0] + agg_ref[1]
    h = jnp.maximum(t @ Wn_ref[...] + rv, 0.0)
    h_ref[...] = h

    @pl.when(pl.program_id(0) == 0)
    def _init():
        hsum_ref[...] = jnp.zeros_like(hsum_ref)

    hsum_ref[...] += jnp.sum(h, axis=0, keepdims=True)


_dense = pl.pallas_call(
    _dense_body,
    grid=(N_BLKS,),
    in_specs=[_row_spec, _agg_spec, _vec_spec, _vec_spec, _vec_spec,
              _mat_spec, _vec_spec, _mat_spec, _mat_spec, _mat_spec],
    out_specs=[_row_spec, _vec_spec],
    out_shape=[jax.ShapeDtypeStruct((N, D), jnp.float32),
               jax.ShapeDtypeStruct((1, D), jnp.float32)],
)


def _dense3_body(x_ref, agg_ref, hg_ref, hi_ref, hl_ref, Wn_ref, bn_ref,
                 Wg_ref, Wi_ref, Wl_ref, Wfc1_ref, bfc1_ref, Wfc2_ref,
                 bfc2_ref, out_ref, hsum_ref):
    rv = (bn_ref[...] + hg_ref[...] @ Wg_ref[...] + hi_ref[...] @ Wi_ref[...]
          + hl_ref[...] @ Wl_ref[...])
    t = x_ref[...] + agg_ref[0] + agg_ref[1]
    h = jnp.maximum(t @ Wn_ref[...] + rv, 0.0)

    @pl.when(pl.program_id(0) == 0)
    def _init():
        hsum_ref[...] = jnp.zeros_like(hsum_ref)

    hsum_ref[...] += jnp.sum(h, axis=0, keepdims=True)

    @pl.when(pl.program_id(0) == N_BLKS - 1)
    def _head():
        z = jnp.maximum(hsum_ref[...] @ Wfc1_ref[...] + bfc1_ref[...], 0.0)
        out_ref[...] = z @ Wfc2_ref[...] + bfc2_ref[...]


_dense3 = pl.pallas_call(
    _dense3_body,
    grid=(N_BLKS,),
    in_specs=[_row_spec, _agg_spec, _vec_spec, _vec_spec, _vec_spec,
              _mat_spec, _vec_spec, _mat_spec, _mat_spec, _mat_spec,
              _mat_spec, _vec_spec, pl.BlockSpec((D, A), lambda i: (0, 0)),
              pl.BlockSpec((1, A), lambda i: (0, 0))],
    out_specs=pl.BlockSpec((1, A), lambda i: (0, 0)),
    out_shape=jax.ShapeDtypeStruct((1, A), jnp.float32),
    scratch_shapes=[pltpu.VMEM((1, D), jnp.float32)],
)


def kernel(x, edge_index, x_initial, x_lead, W1n, b1n, W1g, W1i, W1l, W2n,
           b2n, W2g, W2i, W2l, W3n, b3n, W3g, W3i, W3l, Wfc1, bfc1, Wfc2,
           bfc2):
    src = edge_index[0]
    dst = edge_index[1]
    pad = E_PAD - E
    src_p = jnp.concatenate([src, jnp.zeros((pad,), jnp.int32)])
    dst_p = jnp.concatenate([dst, jnp.full((pad,), N, jnp.int32)])
    src_p = src_p.reshape(NW, K, CH)
    dst_p = dst_p.reshape(NW, K, CH)

    b1n_ = b1n.reshape(1, D)
    b2n_ = b2n.reshape(1, D)
    b3n_ = b3n.reshape(1, D)
    bfc1_ = bfc1.reshape(1, D)
    bfc2_ = bfc2.reshape(1, A)

    sx, si, sl = _colsum3(x, x_initial, x_lead)

    agg1 = _sc_segsum(x, src_p, dst_p)
    h1, s1 = _dense(x, agg1, sx, si, sl, W1n, b1n_, W1g, W1i, W1l)
    agg2 = _sc_segsum(h1, src_p, dst_p)
    h2, s2 = _dense(h1, agg2, s1, si, sl, W2n, b2n_, W2g, W2i, W2l)
    agg3 = _sc_segsum(h2, src_p, dst_p)
    return _dense3(h2, agg3, s2, si, sl, W3n, b3n_, W3g, W3i, W3l, Wfc1,
                   bfc1_, Wfc2, bfc2_)


# speedup vs baseline: 3.2371x; 1.0000x over previous
import functools

import jax
import jax.numpy as jnp
from jax import lax
from jax.experimental import pallas as pl
from jax.experimental.pallas import tpu as pltpu
from jax.experimental.pallas import tpu_sc as plsc

N = 10000
E = 320000
D = 128
A = 64

NC = 2
NS = 16
NW = NC * NS
CH = 128
K = 80
KB = 40
NKB = K // KB
NBUF = 2
E_PAD = NW * K * CH
N_ACC = 10240

ROWS_BLK = 2000
N_BLKS = N // ROWS_BLK

_sc_mesh = plsc.VectorSubcoreMesh(
    core_axis_name="c", subcore_axis_name="s", num_cores=NC, num_subcores=NS)


@functools.partial(
    pl.kernel,
    out_type=jax.ShapeDtypeStruct((NC, N_ACC, D), jnp.float32),
    mesh=_sc_mesh,
    scratch_types=[
        pltpu.MemorySpace.VMEM_SHARED((N_ACC, D), jnp.float32),
        pltpu.VMEM((KB, CH), jnp.int32),
        pltpu.VMEM((KB, CH), jnp.int32),
        pltpu.VMEM((CH, D), jnp.float32),
        pltpu.VMEM((CH, D), jnp.float32),
        pltpu.SemaphoreType.DMA,
        pltpu.SemaphoreType.DMA,
    ],
)
def _sc_segsum(table, src_idx, dst_idx, out, acc, src_v, dst_v, gbuf0, gbuf1,
               sem0, sem1):
    gbufs = (gbuf0, gbuf1)
    rows = gbuf0
    c = lax.axis_index("c")
    s = lax.axis_index("s")
    w = c * NS + s
    sems = (sem0, sem1)

    zero = jnp.zeros((16,), jnp.float32)

    def _zrow(i, carry):
        for jj in range(D // 16):
            rows[i, pl.ds(jj * 16, 16)] = zero
        return carry

    lax.fori_loop(0, CH, _zrow, 0)
    for b in range(640 // CH):
        pltpu.sync_copy(rows, acc.at[pl.ds(s * 640 + b * CH, CH)])
    plsc.subcore_barrier()

    def _wait(sem, q):
        pltpu.make_async_copy(table.at[pl.ds(0, CH)], gbufs[q], sem).wait()

    for kb in range(NKB):
        pltpu.sync_copy(src_idx.at[w].at[pl.ds(kb * KB, KB)], src_v)
        pltpu.sync_copy(dst_idx.at[w].at[pl.ds(kb * KB, KB)], dst_v)

        for q in range(NBUF):
            pltpu.async_copy(table.at[src_v.at[q]], gbufs[q], sems[q])

        def _round(t, carry2):
            j = t * NBUF
            for q in range(NBUF):
                _wait(sems[q], q)
                pltpu.sync_copy(gbufs[q], acc.at[dst_v.at[j + q]], add=True)

                @pl.when(j + q + NBUF < KB)
                def _arm(q=q):
                    pltpu.async_copy(table.at[src_v.at[j + q + NBUF]],
                                     gbufs[q], sems[q])
            return carry2

        lax.fori_loop(0, KB // NBUF, _round, 0)
    plsc.subcore_barrier()

    pltpu.sync_copy(acc.at[pl.ds(s * 640, 640)],
                    out.at[c].at[pl.ds(s * 640, 640)])


def _colsum3_body(x_ref, xi_ref, xl_ref, sx_ref, si_ref, sl_ref):
    @pl.when(pl.program_id(0) == 0)
    def _init():
        sx_ref[...] = jnp.zeros_like(sx_ref)
        si_ref[...] = jnp.zeros_like(si_ref)
        sl_ref[...] = jnp.zeros_like(sl_ref)

    sx_ref[...] += jnp.sum(x_ref[...], axis=0, keepdims=True)
    si_ref[...] += jnp.sum(xi_ref[...], axis=0, keepdims=True)
    sl_ref[...] += jnp.sum(xl_ref[...], axis=0, keepdims=True)


_row_spec = pl.BlockSpec((ROWS_BLK, D), lambda i: (i, 0))
_vec_spec = pl.BlockSpec((1, D), lambda i: (0, 0))
_mat_spec = pl.BlockSpec((D, D), lambda i: (0, 0))
_agg_spec = pl.BlockSpec((NC, ROWS_BLK, D), lambda i: (0, i, 0))

_colsum3 = pl.pallas_call(
    _colsum3_body,
    grid=(N_BLKS,),
    in_specs=[_row_spec, _row_spec, _row_spec],
    out_specs=[_vec_spec, _vec_spec, _vec_spec],
    out_shape=[jax.ShapeDtypeStruct((1, D), jnp.float32)] * 3,
)


def _dense_body(x_ref, agg_ref, hg_ref, hi_ref, hl_ref, Wn_ref, bn_ref,
                Wg_ref, Wi_ref, Wl_ref, h_ref, hsum_ref):
    rv = (bn_ref[...] + hg_ref[...] @ Wg_ref[...] + hi_ref[...] @ Wi_ref[...]
          + hl_ref[...] @ Wl_ref[...])
    t = x_ref[...] + agg_ref[0] + agg_ref[1]
    h = jnp.maximum(t @ Wn_ref[...] + rv, 0.0)
    h_ref[...] = h

    @pl.when(pl.program_id(0) == 0)
    def _init():
        hsum_ref[...] = jnp.zeros_like(hsum_ref)

    hsum_ref[...] += jnp.sum(h, axis=0, keepdims=True)


_dense = pl.pallas_call(
    _dense_body,
    grid=(N_BLKS,),
    in_specs=[_row_spec, _agg_spec, _vec_spec, _vec_spec, _vec_spec,
              _mat_spec, _vec_spec, _mat_spec, _mat_spec, _mat_spec],
    out_specs=[_row_spec, _vec_spec],
    out_shape=[jax.ShapeDtypeStruct((N, D), jnp.float32),
               jax.ShapeDtypeStruct((1, D), jnp.float32)],
)


def _dense3_body(x_ref, agg_ref, hg_ref, hi_ref, hl_ref, Wn_ref, bn_ref,
                 Wg_ref, Wi_ref, Wl_ref, Wfc1_ref, bfc1_ref, Wfc2_ref,
                 bfc2_ref, out_ref, hsum_ref):
    rv = (bn_ref[...] + hg_ref[...] @ Wg_ref[...] + hi_ref[...] @ Wi_ref[...]
          + hl_ref[...] @ Wl_ref[...])
    t = x_ref[...] + agg_ref[0] + agg_ref[1]
    h = jnp.maximum(t @ Wn_ref[...] + rv, 0.0)

    @pl.when(pl.program_id(0) == 0)
    def _init():
        hsum_ref[...] = jnp.zeros_like(hsum_ref)

    hsum_ref[...] += jnp.sum(h, axis=0, keepdims=True)

    @pl.when(pl.program_id(0) == N_BLKS - 1)
    def _head():
        z = jnp.maximum(hsum_ref[...] @ Wfc1_ref[...] + bfc1_ref[...], 0.0)
        out_ref[...] = z @ Wfc2_ref[...] + bfc2_ref[...]


_dense3 = pl.pallas_call(
    _dense3_body,
    grid=(N_BLKS,),
    in_specs=[_row_spec, _agg_spec, _vec_spec, _vec_spec, _vec_spec,
              _mat_spec, _vec_spec, _mat_spec, _mat_spec, _mat_spec,
              _mat_spec, _vec_spec, pl.BlockSpec((D, A), lambda i: (0, 0)),
              pl.BlockSpec((1, A), lambda i: (0, 0))],
    out_specs=pl.BlockSpec((1, A), lambda i: (0, 0)),
    out_shape=jax.ShapeDtypeStruct((1, A), jnp.float32),
    scratch_shapes=[pltpu.VMEM((1, D), jnp.float32)],
)


def _as_table(hb):
    return hb


def kernel(x, edge_index, x_initial, x_lead, W1n, b1n, W1g, W1i, W1l, W2n,
           b2n, W2g, W2i, W2l, W3n, b3n, W3g, W3i, W3l, Wfc1, bfc1, Wfc2,
           bfc2):
    src = edge_index[0]
    dst = edge_index[1]
    pad = E_PAD - E
    src_p = jnp.concatenate([src, jnp.zeros((pad,), jnp.int32)])
    dst_p = jnp.concatenate([dst, jnp.full((pad,), N, jnp.int32)])
    src_p = src_p.reshape(NW, K, CH)
    dst_p = dst_p.reshape(NW, K, CH)

    b1n_ = b1n.reshape(1, D)
    b2n_ = b2n.reshape(1, D)
    b3n_ = b3n.reshape(1, D)
    bfc1_ = bfc1.reshape(1, D)
    bfc2_ = bfc2.reshape(1, A)

    sx, si, sl = _colsum3(x, x_initial, x_lead)

    agg1 = _sc_segsum(x, src_p, dst_p)
    h1, s1 = _dense(x, agg1, sx, si, sl, W1n, b1n_, W1g, W1i, W1l)
    agg2 = _sc_segsum(h1, src_p, dst_p)
    h2, s2 = _dense(h1, agg2, s1, si, sl, W2n, b2n_, W2g, W2i, W2l)
    agg3 = _sc_segsum(h2, src_p, dst_p)
    return _dense3(h2, agg3, s2, si, sl, W3n, b3n_, W3g, W3i, W3l, Wfc1,
                   bfc1_, Wfc2, bfc2_)


# speedup vs baseline: 3.2372x; 1.0000x over previous
import functools

import jax
import jax.numpy as jnp
from jax import lax
from jax.experimental import pallas as pl
from jax.experimental.pallas import tpu as pltpu
from jax.experimental.pallas import tpu_sc as plsc

N = 10000
E = 320000
D = 128
A = 64

NC = 2
NS = 16
NW = NC * NS
CH = 128
K = 80
KB = 40
NKB = K // KB
NBUF = 2
E_PAD = NW * K * CH
N_ACC = 10240

ROWS_BLK = 2000
N_BLKS = N // ROWS_BLK

_sc_mesh = plsc.VectorSubcoreMesh(
    core_axis_name="c", subcore_axis_name="s", num_cores=NC, num_subcores=NS)


@functools.partial(
    pl.kernel,
    out_type=jax.ShapeDtypeStruct((NC, N_ACC, D), jnp.float32),
    mesh=_sc_mesh,
    scratch_types=[
        pltpu.MemorySpace.VMEM_SHARED((N_ACC, D), jnp.float32),
        pltpu.VMEM((KB, CH), jnp.int32),
        pltpu.VMEM((KB, CH), jnp.int32),
        pltpu.VMEM((CH, D), jnp.float32),
        pltpu.VMEM((CH, D), jnp.float32),
        pltpu.SemaphoreType.DMA,
        pltpu.SemaphoreType.DMA,
    ],
)
def _sc_segsum(table, src_idx, dst_idx, out, acc, src_v, dst_v, gbuf0, gbuf1,
               sem0, sem1):
    gbufs = (gbuf0, gbuf1)
    rows = gbuf0
    c = lax.axis_index("c")
    s = lax.axis_index("s")
    w = c * NS + s
    sems = (sem0, sem1)

    zero = jnp.zeros((16,), jnp.float32)

    def _zrow(i, carry):
        for jj in range(D // 16):
            rows[i, pl.ds(jj * 16, 16)] = zero
        return carry

    lax.fori_loop(0, CH, _zrow, 0)
    for b in range(640 // CH):
        pltpu.sync_copy(rows, acc.at[pl.ds(s * 640 + b * CH, CH)])
    plsc.subcore_barrier()

    def _wait(sem, q):
        pltpu.make_async_copy(table.at[pl.ds(0, CH)], gbufs[q], sem).wait()

    for kb in range(NKB):
        pltpu.sync_copy(src_idx.at[w].at[pl.ds(kb * KB, KB)], src_v)
        pltpu.sync_copy(dst_idx.at[w].at[pl.ds(kb * KB, KB)], dst_v)

        for q in range(NBUF):
            pltpu.async_copy(table.at[src_v.at[q]], gbufs[q], sems[q])

        def _round(t, carry2):
            j = t * NBUF
            for q in range(NBUF):
                _wait(sems[q], q)
                pltpu.sync_copy(gbufs[q], acc.at[dst_v.at[j + q]], add=True)

                @pl.when(j + q + NBUF < KB)
                def _arm(q=q):
                    pltpu.async_copy(table.at[src_v.at[j + q + NBUF]],
                                     gbufs[q], sems[q])
            return carry2

        lax.fori_loop(0, KB // NBUF, _round, 0)
    plsc.subcore_barrier()

    pltpu.sync_copy(acc.at[pl.ds(s * 640, 640)],
                    out.at[c].at[pl.ds(s * 640, 640)])


def _colsum3_body(x_ref, xi_ref, xl_ref, sx_ref, si_ref, sl_ref):
    @pl.when(pl.program_id(0) == 0)
    def _init():
        sx_ref[...] = jnp.zeros_like(sx_ref)
        si_ref[...] = jnp.zeros_like(si_ref)
        sl_ref[...] = jnp.zeros_like(sl_ref)

    sx_ref[...] += jnp.sum(x_ref[...], axis=0, keepdims=True)
    si_ref[...] += jnp.sum(xi_ref[...], axis=0, keepdims=True)
    sl_ref[...] += jnp.sum(xl_ref[...], axis=0, keepdims=True)


_row_spec = pl.BlockSpec((ROWS_BLK, D), lambda i: (i, 0))
_vec_spec = pl.BlockSpec((1, D), lambda i: (0, 0))
_mat_spec = pl.BlockSpec((D, D), lambda i: (0, 0))
_agg_spec = pl.BlockSpec((NC, ROWS_BLK, D), lambda i: (0, i, 0))

_colsum3 = pl.pallas_call(
    _colsum3_body,
    grid=(N_BLKS,),
    in_specs=[_row_spec, _row_spec, _row_spec],
    out_specs=[_vec_spec, _vec_spec, _vec_spec],
    out_shape=[jax.ShapeDtypeStruct((1, D), jnp.float32)] * 3,
)


def _dense_body(x_ref, agg_ref, hg_ref, hi_ref, hl_ref, Wn_ref, bn_ref,
                Wg_ref, Wi_ref, Wl_ref, h_ref, hsum_ref):
    rv = (bn_ref[...] + hg_ref[...] @ Wg_ref[...] + hi_ref[...] @ Wi_ref[...]
          + hl_ref[...] @ Wl_ref[...])
    t = x_ref[...] + agg_ref[0] + agg_ref[1]
    h = jnp.maximum(t @ Wn_ref[...] + rv, 0.0)
    h_ref[...] = h

    @pl.when(pl.program_id(0) == 0)
    def _init():
        hsum_ref[...] = jnp.zeros_like(hsum_ref)

    hsum_ref[...] += jnp.sum(h, axis=0, keepdims=True)


_dense = pl.pallas_call(
    _dense_body,
    grid=(N_BLKS,),
    in_specs=[_row_spec, _agg_spec, _vec_spec, _vec_spec, _vec_spec,
              _mat_spec, _vec_spec, _mat_spec, _mat_spec, _mat_spec],
    out_specs=[_row_spec, _vec_spec],
    out_shape=[jax.ShapeDtypeStruct((N, D), jnp.float32),
               jax.ShapeDtypeStruct((1, D), jnp.float32)],
)


def _dense3_body(x_ref, agg_ref, hg_ref, hi_ref, hl_ref, Wn_ref, bn_ref,
                 Wg_ref, Wi_ref, Wl_ref, Wfc1_ref, bfc1_ref, Wfc2_ref,
                 bfc2_ref, out_ref, hsum_ref):
    rv = (bn_ref[...] + hg_ref[...] @ Wg_ref[...] + hi_ref[...] @ Wi_ref[...]
          + hl_ref[...] @ Wl_ref[...])
    t = x_ref[...] + agg_ref[0] + agg_ref[1]
    h = jnp.maximum(t @ Wn_ref[...] + rv, 0.0)

    @pl.when(pl.program_id(0) == 0)
    def _init():
        hsum_ref[...] = jnp.zeros_like(hsum_ref)

    hsum_ref[...] += jnp.sum(h, axis=0, keepdims=True)

    @pl.when(pl.program_id(0) == N_BLKS - 1)
    def _head():
        z = jnp.maximum(hsum_ref[...] @ Wfc1_ref[...] + bfc1_ref[...], 0.0)
        out_ref[...] = z @ Wfc2_ref[...] + bfc2_ref[...]


_dense3 = pl.pallas_call(
    _dense3_body,
    grid=(N_BLKS,),
    in_specs=[_row_spec, _agg_spec, _vec_spec, _vec_spec, _vec_spec,
              _mat_spec, _vec_spec, _mat_spec, _mat_spec, _mat_spec,
              _mat_spec, _vec_spec, pl.BlockSpec((D, A), lambda i: (0, 0)),
              pl.BlockSpec((1, A), lambda i: (0, 0))],
    out_specs=pl.BlockSpec((1, A), lambda i: (0, 0)),
    out_shape=jax.ShapeDtypeStruct((1, A), jnp.float32),
    scratch_shapes=[pltpu.VMEM((1, D), jnp.float32)],
)


def kernel(x, edge_index, x_initial, x_lead, W1n, b1n, W1g, W1i, W1l, W2n,
           b2n, W2g, W2i, W2l, W3n, b3n, W3g, W3i, W3l, Wfc1, bfc1, Wfc2,
           bfc2):
    src = edge_index[0]
    dst = edge_index[1]
    pad = E_PAD - E
    src_p = jnp.concatenate([src, jnp.zeros((pad,), jnp.int32)])
    dst_p = jnp.concatenate([dst, jnp.full((pad,), N, jnp.int32)])
    src_p = src_p.reshape(NW, K, CH)
    dst_p = dst_p.reshape(NW, K, CH)

    b1n_ = b1n.reshape(1, D)
    b2n_ = b2n.reshape(1, D)
    b3n_ = b3n.reshape(1, D)
    bfc1_ = bfc1.reshape(1, D)
    bfc2_ = bfc2.reshape(1, A)

    sx, si, sl = _colsum3(x, x_initial, x_lead)

    agg1 = _sc_segsum(x, src_p, dst_p)
    h1, s1 = _dense(x, agg1, sx, si, sl, W1n, b1n_, W1g, W1i, W1l)
    agg2 = _sc_segsum(h1, src_p, dst_p)
    h2, s2 = _dense(h1, agg2, s1, si, sl, W2n, b2n_, W2g, W2i, W2l)
    agg3 = _sc_segsum(h2, src_p, dst_p)
    return _dense3(h2, agg3, s2, si, sl, W3n, b3n_, W3g, W3i, W3l, Wfc1,
                   bfc1_, Wfc2, bfc2_)
